# Initial kernel scaffold; baseline (speedup 1.0000x reference)
import functools

import jax
import jax.numpy as jnp
from jax import lax
from jax.experimental import pallas as pl
from jax.experimental.pallas import tpu as pltpu
from jax.experimental.pallas import tpu_sc as plsc

_B, _N, _CI, _CO, _K = 8, 2048, 64, 128, 20


_ROWS = 256


def _tree_sum_lanes(s):
    w = s.shape[1]
    while w > 1:
        s = s[:, : w // 2] + s[:, w // 2:]
        w //= 2
    return s


def _tree_sum_sublanes(s):
    h = s.shape[0]
    while h > 1:
        s = s[: h // 2, :] + s[h // 2:, :]
        h //= 2
    return s


def _topk_body(xr_ref, xft_ref, idx_ref):
    xr = xr_ref[0]
    xft = xft_ref[0]
    inner = -2.0 * lax.dot_general(
        xr, xft, (((1,), (0,)), ((), ())),
        preferred_element_type=jnp.float32)
    xx_col = _tree_sum_sublanes(xft * xft)
    xx_row = _tree_sum_lanes(xr * xr)
    pd = (-xx_col - inner) - xx_row

    col = lax.broadcasted_iota(jnp.int32, (_ROWS, _N), 1)
    vals = pd
    picks = []
    for t in range(_K):
        m = jnp.max(vals, axis=1, keepdims=True)
        q = jnp.where(vals == m, col, _N)
        am = jnp.min(q, axis=1, keepdims=True)
        picks.append(am)
        if t < _K - 1:
            vals = jnp.where(q == am, -jnp.inf, vals)
    idx_ref[0] = jnp.concatenate(picks, axis=1)


def _topk_call(x, xt):
    return pl.pallas_call(
        _topk_body,
        grid=(_B, _N // _ROWS),
        in_specs=[
            pl.BlockSpec((1, _ROWS, _CI), lambda b, r: (b, r, 0)),
            pl.BlockSpec((1, _CI, _N), lambda b, r: (b, 0, 0)),
        ],
        out_specs=pl.BlockSpec((1, _ROWS, _K), lambda b, r: (b, r, 0)),
        out_shape=jax.ShapeDtypeStruct((_B, _N, _K), jnp.int32),
    )(x, xt)


_SC_INFO = plsc.get_sparse_core_info()
_NC, _NS = _SC_INFO.num_cores, _SC_INFO.num_subcores
_NW = _NC * _NS
_PAIRS = (_B * _K) // _NW
_CH = 128


def _gather_body(x_hbm, idxt_hbm, xn_hbm, idx_v, rows_v, gsem, ssem):
    c = lax.axis_index("c")
    s = lax.axis_index("s")
    wid = s * _NC + c
    p0 = wid * _PAIRS
    pltpu.sync_copy(idxt_hbm.at[pl.ds(p0, _PAIRS)], idx_v)

    for j in range(_PAIRS):
        p = p0 + j
        b = p // _K
        n_chunks = _N // _CH

        def chunk(i, _):
            src = x_hbm.at[b].at[idx_v.at[j].at[pl.ds(i * _CH, _CH)]]
            pltpu.async_copy(src, rows_v, gsem).wait()
            pltpu.async_copy(rows_v, xn_hbm.at[p].at[pl.ds(i * _CH, _CH)],
                             ssem).wait()
            return 0

        lax.fori_loop(0, n_chunks, chunk, 0)


def _gather_call(x, idxt):
    run = functools.partial(
        pl.kernel,
        mesh=plsc.VectorSubcoreMesh(core_axis_name="c", subcore_axis_name="s"),
        out_type=jax.ShapeDtypeStruct((_B * _K, _N, _CI), jnp.float32),
        scratch_types=[
            pltpu.VMEM((_PAIRS, _N), jnp.int32),
            pltpu.VMEM((_CH, _CI), jnp.float32),
            pltpu.SemaphoreType.DMA,
            pltpu.SemaphoreType.DMA,
        ],
    )(_gather_body)
    return run(x, idxt)


_RC = 512


def _conv_body(xn_ref, x_ref, w1a_ref, wq_ref, b1_ref, w2_ref, b2_ref, out_ref):
    xb = x_ref[0]
    q = jnp.dot(xb, wq_ref[...],
                preferred_element_type=jnp.float32) + b1_ref[...]
    acc = None
    for k in range(_K):
        nb = xn_ref[0, k]
        h = jnp.dot(nb, w1a_ref[...],
                    preferred_element_type=jnp.float32) + q
        h = jnp.where(h >= 0, h, 0.2 * h)
        h = jnp.dot(h, w2_ref[...],
                    preferred_element_type=jnp.float32) + b2_ref[...]
        h = jnp.where(h >= 0, h, 0.2 * h)
        acc = h if acc is None else jnp.maximum(acc, h)
    out_ref[0] = acc


def _conv_call(xn, x, w1a_t, wq_t, b1r, w2_t, b2r):
    return pl.pallas_call(
        _conv_body,
        grid=(_B, _N // _RC),
        in_specs=[
            pl.BlockSpec((1, _K, _RC, _CI), lambda b, r: (b, 0, r, 0)),
            pl.BlockSpec((1, _RC, _CI), lambda b, r: (b, r, 0)),
            pl.BlockSpec((_CI, _CO), lambda b, r: (0, 0)),
            pl.BlockSpec((_CI, _CO), lambda b, r: (0, 0)),
            pl.BlockSpec((1, _CO), lambda b, r: (0, 0)),
            pl.BlockSpec((_CO, _CO), lambda b, r: (0, 0)),
            pl.BlockSpec((1, _CO), lambda b, r: (0, 0)),
        ],
        out_specs=pl.BlockSpec((1, _RC, _CO), lambda b, r: (b, r, 0)),
        out_shape=jax.ShapeDtypeStruct((_B, _N, _CO), jnp.float32),
    )(xn, x, w1a_t, wq_t, b1r, w2_t, b2r)


def kernel(x, W1, b1, W2, b2):
    xt = jnp.swapaxes(x, 2, 1)
    idx = _topk_call(x, xt)
    idxt = jnp.swapaxes(idx, 2, 1).reshape(_B * _K, _N)
    xn = _gather_call(x, idxt).reshape(_B, _K, _N, _CI)
    w1a = W1[:, :_CI]
    wq = W1[:, _CI:] - w1a
    out = _conv_call(xn, x, w1a.T, wq.T, b1.reshape(1, _CO),
                     W2.T, b2.reshape(1, _CO))
    return out, idx


# speedup vs baseline: 11.0721x; 11.0721x over previous
import functools

import jax
import jax.numpy as jnp
from jax import lax
from jax.experimental import pallas as pl
from jax.experimental.pallas import tpu as pltpu
from jax.experimental.pallas import tpu_sc as plsc

_B, _N, _CI, _CO, _K = 8, 2048, 64, 128, 20


_ROWS = 256


def _tree_sum_lanes(s):
    w = s.shape[1]
    while w > 1:
        s = s[:, : w // 2] + s[:, w // 2:]
        w //= 2
    return s


def _tree_sum_sublanes(s):
    h = s.shape[0]
    while h > 1:
        s = s[: h // 2, :] + s[h // 2:, :]
        h //= 2
    return s


def _topk_body(xr_ref, xft_ref, w1a_ref, wq_ref, b1_ref, idx_ref, p_ref, q_ref):
    xr = xr_ref[0]
    xft = xft_ref[0]
    p_ref[0] = jnp.dot(xr, w1a_ref[...], preferred_element_type=jnp.float32)
    q_ref[0] = jnp.dot(xr, wq_ref[...],
                       preferred_element_type=jnp.float32) + b1_ref[...]
    inner = -2.0 * lax.dot_general(
        xr, xft, (((1,), (0,)), ((), ())),
        preferred_element_type=jnp.float32)
    xx_col = _tree_sum_sublanes(xft * xft)
    xx_row = _tree_sum_lanes(xr * xr)
    pd = (-xx_col - inner) - xx_row

    col = lax.broadcasted_iota(jnp.int32, (_ROWS, _N), 1)
    vals = pd
    picks = []
    for t in range(_K):
        m = jnp.max(vals, axis=1, keepdims=True)
        q = jnp.where(vals == m, col, _N)
        am = jnp.min(q, axis=1, keepdims=True)
        picks.append(am)
        if t < _K - 1:
            vals = jnp.where(q == am, -jnp.inf, vals)
    idx_ref[0] = jnp.concatenate(picks, axis=1)


def _topk_call(x, xt, w1a_t, wq_t, b1r):
    return pl.pallas_call(
        _topk_body,
        grid=(_B, _N // _ROWS),
        in_specs=[
            pl.BlockSpec((1, _ROWS, _CI), lambda b, r: (b, r, 0)),
            pl.BlockSpec((1, _CI, _N), lambda b, r: (b, 0, 0)),
            pl.BlockSpec((_CI, _CO), lambda b, r: (0, 0)),
            pl.BlockSpec((_CI, _CO), lambda b, r: (0, 0)),
            pl.BlockSpec((1, _CO), lambda b, r: (0, 0)),
        ],
        out_specs=[
            pl.BlockSpec((1, _ROWS, _K), lambda b, r: (b, r, 0)),
            pl.BlockSpec((1, _ROWS, _CO), lambda b, r: (b, r, 0)),
            pl.BlockSpec((1, _ROWS, _CO), lambda b, r: (b, r, 0)),
        ],
        out_shape=[
            jax.ShapeDtypeStruct((_B, _N, _K), jnp.int32),
            jax.ShapeDtypeStruct((_B, _N, _CO), jnp.float32),
            jax.ShapeDtypeStruct((_B, _N, _CO), jnp.float32),
        ],
    )(x, xt, w1a_t, wq_t, b1r)


_NC, _NS = 2, 16
_NW = _NC * _NS
_PAIRS = (_B * _K) // _NW
_CH = 128


def _gather_body(p_hbm, idxf_hbm, pn_hbm, idx_v, rows_v, gsem, ssem):
    c = lax.axis_index("c")
    s = lax.axis_index("s")
    wid = s * _NC + c
    p0 = wid * _PAIRS

    for j in range(_PAIRS):
        p = p0 + j
        b = p // _K
        start = pl.multiple_of(p * _N, 256)
        pltpu.sync_copy(idxf_hbm.at[pl.ds(start, _N)], idx_v)
        n_chunks = _N // _CH

        def chunk(i, _):
            off = pl.multiple_of(i * _CH, _CH)
            src = p_hbm.at[b].at[idx_v.at[pl.ds(off, _CH)]]
            pltpu.async_copy(src, rows_v, gsem).wait()
            pltpu.async_copy(rows_v, pn_hbm.at[p].at[pl.ds(off, _CH)],
                             ssem).wait()
            return 0

        lax.fori_loop(0, n_chunks, chunk, 0)


def _gather_call(p, idxf):
    run = functools.partial(
        pl.kernel,
        mesh=plsc.VectorSubcoreMesh(core_axis_name="c", subcore_axis_name="s"),
        out_type=jax.ShapeDtypeStruct((_B * _K, _N, _CO), jnp.float32),
        scratch_types=[
            pltpu.VMEM((_N,), jnp.int32),
            pltpu.VMEM((_CH, _CO), jnp.float32),
            pltpu.SemaphoreType.DMA,
            pltpu.SemaphoreType.DMA,
        ],
    )(_gather_body)
    return run(p, idxf)


_RC = 512


def _conv_body(pn_ref, q_ref, w2_ref, b2_ref, out_ref):
    q = q_ref[0]
    acc = None
    for k in range(_K):
        h = pn_ref[0, k] + q
        h = jnp.where(h >= 0, h, 0.2 * h)
        h = jnp.dot(h, w2_ref[...],
                    preferred_element_type=jnp.float32) + b2_ref[...]
        h = jnp.where(h >= 0, h, 0.2 * h)
        acc = h if acc is None else jnp.maximum(acc, h)
    out_ref[0] = acc


def _conv_call(pn, q, w2_t, b2r):
    return pl.pallas_call(
        _conv_body,
        grid=(_B, _N // _RC),
        in_specs=[
            pl.BlockSpec((1, _K, _RC, _CO), lambda b, r: (b, 0, r, 0)),
            pl.BlockSpec((1, _RC, _CO), lambda b, r: (b, r, 0)),
            pl.BlockSpec((_CO, _CO), lambda b, r: (0, 0)),
            pl.BlockSpec((1, _CO), lambda b, r: (0, 0)),
        ],
        out_specs=pl.BlockSpec((1, _RC, _CO), lambda b, r: (b, r, 0)),
        out_shape=jax.ShapeDtypeStruct((_B, _N, _CO), jnp.float32),
    )(pn, q, w2_t, b2r)


def kernel(x, W1, b1, W2, b2):
    xt = jnp.swapaxes(x, 2, 1)
    w1a = W1[:, :_CI]
    wq = W1[:, _CI:] - w1a
    idx, p, q = _topk_call(x, xt, w1a.T, wq.T, b1.reshape(1, _CO))
    idxf = jnp.swapaxes(idx, 2, 1).reshape(_B * _K * _N)
    pn = _gather_call(p, idxf).reshape(_B, _K, _N, _CO)
    out = _conv_call(pn, q, W2.T, b2.reshape(1, _CO))
    return out, idx
